# baseline (speedup 1.0000x reference)
import functools

import jax
import jax.numpy as jnp
from jax import lax
from jax.experimental import pallas as pl
from jax.experimental.pallas import tpu as pltpu
from jax.experimental.pallas import tpu_sc as plsc

N = 10000
D = 128
E = 320000
NREL = 6

NC = 2
NS = 16
NW = NC * NS
IDXB = 128
ROWS_PER_W = 80
EDGES_PER_W = ROWS_PER_W * IDXB
E_PAD = NW * EDGES_PER_W
GROUP_ROWS = 4
GROUP_E = GROUP_ROWS * IDXB
N_GROUPS = ROWS_PER_W // GROUP_ROWS
SGROUP_ROWS = 2
SGROUP_E = SGROUP_ROWS * IDXB
SN_GROUPS = ROWS_PER_W // SGROUP_ROWS
ZROWS = 64

TE = 2048
TN = 2000
N_PAD = 10240
NPW = N_PAD // NS


def _node_transform_body(h_ref, wa_ref, ga_ref, s_ref, zd_ref):
    x = h_ref[...]
    s_ref[...] = jnp.dot(
        x, wa_ref[...], preferred_element_type=jnp.float32).astype(
            jnp.bfloat16)
    zd_ref[...] = jnp.dot(
        x, ga_ref[...], preferred_element_type=jnp.float32).astype(
            jnp.bfloat16)


def _node_transform(h, wa, ga):
    return pl.pallas_call(
        _node_transform_body,
        grid=(N // TN,),
        in_specs=[
            pl.BlockSpec((TN, D), lambda i: (i, 0)),
            pl.BlockSpec((D, D), lambda i: (0, 0)),
            pl.BlockSpec((D, D), lambda i: (0, 0)),
        ],
        out_specs=[
            pl.BlockSpec((TN, D), lambda i: (i, 0)),
            pl.BlockSpec((TN, D), lambda i: (i, 0)),
        ],
        out_shape=[
            jax.ShapeDtypeStruct((N, D), jnp.bfloat16),
            jax.ShapeDtypeStruct((N, D), jnp.bfloat16),
        ],
    )(h, wa, ga)


def _gelu(x):
    return 0.5 * x * (1.0 + lax.erf(x * 0.7071067811865476))


def _edge_mlp_body(src_ref, dst_ref, rel_ref, reltab_ref, w2_ref, b2_ref,
                   g1b_ref, g2_ref, gb2_ref, out_ref):
    pid = pl.program_id(0)
    rel = rel_ref[0, 0, :]
    onehot = jnp.where(
        lax.broadcasted_iota(jnp.int32, (TE, 8), 1) == rel[:, None],
        1.0, 0.0).astype(jnp.float32)
    relb = jnp.dot(onehot, reltab_ref[...],
                   preferred_element_type=jnp.float32)
    hid = _gelu(src_ref[...].astype(jnp.float32) + relb[:, :D])
    msg = jnp.dot(hid, w2_ref[...],
                  preferred_element_type=jnp.float32) + b2_ref[...]
    g1 = dst_ref[...].astype(jnp.float32) \
        + jnp.dot(msg, g1b_ref[...], preferred_element_type=jnp.float32) \
        + relb[:, D:]
    ghid = _gelu(g1)
    glog = jnp.sum(ghid * g2_ref[...], axis=1, keepdims=True) + gb2_ref[0, 0]
    out = jax.nn.sigmoid(glog) * msg
    eid = pid * TE + lax.broadcasted_iota(jnp.int32, (TE, 1), 0)
    out_ref[...] = jnp.where(eid < E, out, 0.0)


def _edge_mlp(src_g, dst_g, rel3d, reltab, w2, b2, g1b, g2row, gb2):
    return pl.pallas_call(
        _edge_mlp_body,
        grid=(E_PAD // TE,),
        in_specs=[
            pl.BlockSpec((TE, D), lambda i: (i, 0)),
            pl.BlockSpec((TE, D), lambda i: (i, 0)),
            pl.BlockSpec((1, 1, TE), lambda i: (i, 0, 0)),
            pl.BlockSpec((8, 2 * D), lambda i: (0, 0)),
            pl.BlockSpec((D, D), lambda i: (0, 0)),
            pl.BlockSpec((1, D), lambda i: (0, 0)),
            pl.BlockSpec((D, D), lambda i: (0, 0)),
            pl.BlockSpec((1, D), lambda i: (0, 0)),
            pl.BlockSpec((1, 1), lambda i: (0, 0), memory_space=pltpu.SMEM),
        ],
        out_specs=pl.BlockSpec((TE, D), lambda i: (i, 0)),
        out_shape=jax.ShapeDtypeStruct((E_PAD, D), jnp.float32),
    )(src_g, dst_g, rel3d, reltab, w2, b2, g1b, g2row, gb2)


def _ln_body(h_ref, p0_ref, p1_ref, g_ref, b_ref, out_ref):
    x = h_ref[...] + p0_ref[...] + p1_ref[...]
    m = jnp.mean(x, axis=-1, keepdims=True)
    xc = x - m
    v = jnp.mean(xc * xc, axis=-1, keepdims=True)
    out_ref[...] = xc * lax.rsqrt(v + 1e-5) * g_ref[...] + b_ref[...]


def _residual_ln(h, p0, p1, gamma, beta):
    return pl.pallas_call(
        _ln_body,
        grid=(N // TN,),
        in_specs=[
            pl.BlockSpec((TN, D), lambda i: (i, 0)),
            pl.BlockSpec((TN, D), lambda i: (i, 0)),
            pl.BlockSpec((TN, D), lambda i: (i, 0)),
            pl.BlockSpec((1, D), lambda i: (0, 0)),
            pl.BlockSpec((1, D), lambda i: (0, 0)),
        ],
        out_specs=pl.BlockSpec((TN, D), lambda i: (i, 0)),
        out_shape=jax.ShapeDtypeStruct((N, D), jnp.float32),
    )(h, p0, p1, gamma, beta)


@functools.cache
def _sc_mesh():
    return plsc.VectorSubcoreMesh(core_axis_name="c", subcore_axis_name="s",
                                  num_cores=NC, num_subcores=NS)


KBUF = 10

FAST_CID = 1
ROWS_FAST = 80
ROWS_SLOW = 80
assert NS * (ROWS_FAST + ROWS_SLOW) == E_PAD // IDXB


def _gather_kernel_body(s_hbm, zd_hbm, sidx_hbm, didx_hbm, srcg_hbm,
                        dstg_hbm, idxall_v, rows_v, semg, semw):
    cid = lax.axis_index("c")
    sid = lax.axis_index("s")
    is_fast = cid == FAST_CID
    row0 = jnp.where(is_fast, sid * ROWS_FAST,
                     NS * ROWS_FAST + sid * ROWS_SLOW)
    nrows = jnp.where(is_fast, ROWS_FAST, ROWS_SLOW)
    e0 = row0 * IDXB

    def run(table_hbm, idx2d_hbm, out_hbm):
        pltpu.sync_copy(idx2d_hbm.at[pl.ds(row0, ROWS_FAST)], idxall_v)

        def fire_gather(step, b):
            pltpu.async_copy(table_hbm.at[idxall_v.at[step]],
                             rows_v.at[b], semg.at[b])

        def wait_gather(b):
            pltpu.make_async_copy(table_hbm.at[pl.ds(0, IDXB)],
                                  rows_v.at[b], semg.at[b]).wait()

        def fire_wb(step, b):
            pltpu.async_copy(rows_v.at[b],
                             out_hbm.at[pl.ds(e0 + step * IDXB, IDXB)],
                             semw.at[b])

        def wait_wb(b):
            pltpu.make_async_copy(out_hbm.at[pl.ds(0, IDXB)],
                                  rows_v.at[b], semw.at[b]).wait()

        for b in range(KBUF):
            fire_gather(b, b)

        def body(k, carry):
            for j in range(KBUF):
                s = k * KBUF + j
                wait_gather(j)
                fire_wb(s, j)
                jp = (j - 1) % KBUF
                sp = s - 1

                @pl.when((sp >= 0) & (sp + KBUF < nrows))
                def _():
                    wait_wb(jp)
                    fire_gather(sp + KBUF, jp)
            return carry
        lax.fori_loop(0, nrows // KBUF, body, 0)

        for b in range(KBUF):
            wait_wb(b)

    run(s_hbm, sidx_hbm, srcg_hbm)
    run(zd_hbm, didx_hbm, dstg_hbm)


@functools.cache
def _sc_gather_kernel():
    return pl.kernel(
        _gather_kernel_body,
        out_type=[
            jax.ShapeDtypeStruct((E_PAD, D), jnp.bfloat16),
            jax.ShapeDtypeStruct((E_PAD, D), jnp.bfloat16),
        ],
        mesh=_sc_mesh(),
        scratch_types=[
            pltpu.VMEM((ROWS_FAST, IDXB), jnp.int32),
            pltpu.VMEM((KBUF, IDXB, D), jnp.bfloat16),
            pltpu.SemaphoreType.DMA((KBUF,)),
            pltpu.SemaphoreType.DMA((KBUF,)),
        ],
        compiler_params=pltpu.CompilerParams(use_tc_tiling_on_sc=False),
    )


def _sc_gather(s, zd, sidx, didx):
    return _sc_gather_kernel()(s, zd, sidx, didx)


def _scatter_kernel_body(eout_hbm, didx_hbm, out0_hbm, out1_hbm, idx_v,
                         vals_v, zbuf_v, acc_shared):
    cid = lax.axis_index("c")
    sid = lax.axis_index("s")
    wid = sid * NC + cid
    row0 = wid * ROWS_PER_W
    e0 = wid * EDGES_PER_W

    def zrow(i, carry):
        for j in range(D // 16):
            zbuf_v[i, pl.ds(j * 16, 16)] = jnp.zeros((16,), jnp.float32)
        return carry
    lax.fori_loop(0, ZROWS, zrow, 0)
    for t in range(NPW // ZROWS):
        pltpu.sync_copy(zbuf_v,
                        acc_shared.at[pl.ds(sid * NPW + t * ZROWS, ZROWS)])
    plsc.subcore_barrier()

    def group(g, carry):
        r = row0 + g * SGROUP_ROWS
        pltpu.sync_copy(didx_hbm.at[pl.ds(r, SGROUP_ROWS)], idx_v)
        pltpu.sync_copy(eout_hbm.at[pl.ds(e0 + g * SGROUP_E, SGROUP_E)],
                        vals_v)
        for j in range(SGROUP_ROWS):
            pltpu.sync_copy(vals_v.at[pl.ds(j * IDXB, IDXB)],
                            acc_shared.at[idx_v.at[j]], add=True)
        return carry
    lax.fori_loop(0, SN_GROUPS, group, 0)
    plsc.subcore_barrier()

    @pl.when(cid == 0)
    def _():
        pltpu.sync_copy(acc_shared.at[pl.ds(sid * NPW, NPW)],
                        out0_hbm.at[pl.ds(sid * NPW, NPW)])

    @pl.when(cid == 1)
    def _():
        pltpu.sync_copy(acc_shared.at[pl.ds(sid * NPW, NPW)],
                        out1_hbm.at[pl.ds(sid * NPW, NPW)])


@functools.cache
def _sc_scatter_kernel():
    return pl.kernel(
        _scatter_kernel_body,
        out_type=[
            jax.ShapeDtypeStruct((N_PAD, D), jnp.float32),
            jax.ShapeDtypeStruct((N_PAD, D), jnp.float32),
        ],
        mesh=_sc_mesh(),
        scratch_types=[
            pltpu.VMEM((SGROUP_ROWS, IDXB), jnp.int32),
            pltpu.VMEM((SGROUP_E, D), jnp.float32),
            pltpu.VMEM((ZROWS, D), jnp.float32),
            pltpu.VMEM_SHARED((N_PAD, D), jnp.float32),
        ],
        compiler_params=pltpu.CompilerParams(use_tc_tiling_on_sc=False),
    )


def _sc_scatter(eout, didx):
    return _sc_scatter_kernel()(eout, didx)


def kernel(node_states, edge_index, rel_idx, rel_emb, msg_W1, msg_b1,
           msg_W2, msg_b2, gate_W1, gate_b1, gate_W2, gate_b2,
           ln_gamma, ln_beta):
    L = msg_W1.shape[0]

    pad = E_PAD - E
    sidx = jnp.pad(edge_index[0], (0, pad)).reshape(E_PAD // IDXB, IDXB)
    sidx = jnp.pad(sidx, ((0, ROWS_FAST), (0, 0)))
    didx = jnp.pad(edge_index[1], (0, pad)).reshape(E_PAD // IDXB, IDXB)
    didx = jnp.pad(didx, ((0, ROWS_FAST), (0, 0)))
    rel3d = jnp.pad(rel_idx, (0, pad)).reshape(E_PAD // TE, 1, TE)

    h = node_states
    for l in range(L):
        wa = msg_W1[l, :D]
        ga = gate_W1[l, :D]
        rel1 = rel_emb @ msg_W1[l, D:] + msg_b1[l]
        relg = rel_emb @ gate_W1[l, 2 * D:] + gate_b1[l]
        reltab = jnp.zeros((8, 2 * D), jnp.float32)
        reltab = reltab.at[:NREL, :D].set(rel1).at[:NREL, D:].set(relg)
        w2 = msg_W2[l]
        b2 = msg_b2[l][None, :]
        g1b = gate_W1[l, D:2 * D]
        g2row = gate_W2[l][:, 0][None, :]
        gb2 = gate_b2[l][None, :]

        s, zd = _node_transform(h, wa, ga)
        src_g, dst_g = _sc_gather(s, zd, sidx, didx)
        eout = _edge_mlp(src_g, dst_g, rel3d, reltab, w2, b2, g1b, g2row,
                         gb2)
        p0, p1 = _sc_scatter(eout, didx)
        h = _residual_ln(h, p0, p1, ln_gamma[l][None, :],
                         ln_beta[l][None, :])
    return h


# speedup vs baseline: 1.3382x; 1.3382x over previous
import functools

import jax
import jax.numpy as jnp
from jax import lax
from jax.experimental import pallas as pl
from jax.experimental.pallas import tpu as pltpu
from jax.experimental.pallas import tpu_sc as plsc

N = 10000
D = 128
E = 320000
NREL = 6

NC = 2
NS = 16
NW = NC * NS
IDXB = 128
ROWS_PER_W = 80
EDGES_PER_W = ROWS_PER_W * IDXB
E_PAD = NW * EDGES_PER_W
GROUP_ROWS = 4
GROUP_E = GROUP_ROWS * IDXB
N_GROUPS = ROWS_PER_W // GROUP_ROWS
SGROUP_ROWS = 2
SGROUP_E = SGROUP_ROWS * IDXB
SN_GROUPS = ROWS_PER_W // SGROUP_ROWS
ZROWS = 64

TE = 2048
TN = 2000
N_PAD = 10240
NPW = N_PAD // NS


def _node_transform_body(h_ref, wa_ref, ga_ref, s_ref, zd_ref):
    x = h_ref[...]
    s_ref[...] = jnp.dot(
        x, wa_ref[...], preferred_element_type=jnp.float32).astype(
            jnp.bfloat16)
    zd_ref[...] = jnp.dot(
        x, ga_ref[...], preferred_element_type=jnp.float32).astype(
            jnp.bfloat16)


def _node_transform(h, wa, ga):
    return pl.pallas_call(
        _node_transform_body,
        grid=(N // TN,),
        in_specs=[
            pl.BlockSpec((TN, D), lambda i: (i, 0)),
            pl.BlockSpec((D, D), lambda i: (0, 0)),
            pl.BlockSpec((D, D), lambda i: (0, 0)),
        ],
        out_specs=[
            pl.BlockSpec((TN, D), lambda i: (i, 0)),
            pl.BlockSpec((TN, D), lambda i: (i, 0)),
        ],
        out_shape=[
            jax.ShapeDtypeStruct((N, D), jnp.bfloat16),
            jax.ShapeDtypeStruct((N, D), jnp.bfloat16),
        ],
    )(h, wa, ga)


def _gelu(x):
    return 0.5 * x * (1.0 + lax.erf(x * 0.7071067811865476))


def _edge_mlp_body(src_ref, dst_ref, rel_ref, reltab_ref, w2_ref, b2_ref,
                   g1b_ref, g2_ref, gb2_ref, out_ref):
    pid = pl.program_id(0)
    rel = rel_ref[0, 0, :]
    onehot = jnp.where(
        lax.broadcasted_iota(jnp.int32, (TE, 8), 1) == rel[:, None],
        1.0, 0.0).astype(jnp.float32)
    relb = jnp.dot(onehot, reltab_ref[...],
                   preferred_element_type=jnp.float32)
    hid = _gelu(src_ref[...].astype(jnp.float32) + relb[:, :D])
    msg = jnp.dot(hid, w2_ref[...],
                  preferred_element_type=jnp.float32) + b2_ref[...]
    g1 = dst_ref[...].astype(jnp.float32) \
        + jnp.dot(msg, g1b_ref[...], preferred_element_type=jnp.float32) \
        + relb[:, D:]
    ghid = _gelu(g1)
    glog = jnp.sum(ghid * g2_ref[...], axis=1, keepdims=True) + gb2_ref[0, 0]
    out = jax.nn.sigmoid(glog) * msg
    eid = pid * TE + lax.broadcasted_iota(jnp.int32, (TE, 1), 0)
    out_ref[...] = jnp.where(eid < E, out, 0.0)


def _edge_mlp(src_g, dst_g, rel3d, reltab, w2, b2, g1b, g2row, gb2):
    return pl.pallas_call(
        _edge_mlp_body,
        grid=(E_PAD // TE,),
        in_specs=[
            pl.BlockSpec((TE, D), lambda i: (i, 0)),
            pl.BlockSpec((TE, D), lambda i: (i, 0)),
            pl.BlockSpec((1, 1, TE), lambda i: (i, 0, 0)),
            pl.BlockSpec((8, 2 * D), lambda i: (0, 0)),
            pl.BlockSpec((D, D), lambda i: (0, 0)),
            pl.BlockSpec((1, D), lambda i: (0, 0)),
            pl.BlockSpec((D, D), lambda i: (0, 0)),
            pl.BlockSpec((1, D), lambda i: (0, 0)),
            pl.BlockSpec((1, 1), lambda i: (0, 0), memory_space=pltpu.SMEM),
        ],
        out_specs=pl.BlockSpec((TE, D), lambda i: (i, 0)),
        out_shape=jax.ShapeDtypeStruct((E_PAD, D), jnp.float32),
    )(src_g, dst_g, rel3d, reltab, w2, b2, g1b, g2row, gb2)


def _ln_body(h_ref, p0_ref, p1_ref, g_ref, b_ref, out_ref):
    x = h_ref[...] + p0_ref[...] + p1_ref[...]
    m = jnp.mean(x, axis=-1, keepdims=True)
    xc = x - m
    v = jnp.mean(xc * xc, axis=-1, keepdims=True)
    out_ref[...] = xc * lax.rsqrt(v + 1e-5) * g_ref[...] + b_ref[...]


def _residual_ln(h, p0, p1, gamma, beta):
    return pl.pallas_call(
        _ln_body,
        grid=(N // TN,),
        in_specs=[
            pl.BlockSpec((TN, D), lambda i: (i, 0)),
            pl.BlockSpec((TN, D), lambda i: (i, 0)),
            pl.BlockSpec((TN, D), lambda i: (i, 0)),
            pl.BlockSpec((1, D), lambda i: (0, 0)),
            pl.BlockSpec((1, D), lambda i: (0, 0)),
        ],
        out_specs=pl.BlockSpec((TN, D), lambda i: (i, 0)),
        out_shape=jax.ShapeDtypeStruct((N, D), jnp.float32),
    )(h, p0, p1, gamma, beta)


@functools.cache
def _sc_mesh():
    return plsc.VectorSubcoreMesh(core_axis_name="c", subcore_axis_name="s",
                                  num_cores=NC, num_subcores=NS)


KBUF = 4
NSTRIPE = N // NS


def _gather_kernel_body(s_hbm, zd_hbm, sidx_hbm, didx_hbm, srcg_hbm,
                        dstg_hbm, idxall_v, rows_v, tbl_s, tbl_z,
                        semg, semw):
    cid = lax.axis_index("c")
    sid = lax.axis_index("s")
    row0 = (cid * NS + sid) * ROWS_PER_W
    e0 = row0 * IDXB

    pltpu.sync_copy(s_hbm.at[pl.ds(sid * NSTRIPE, NSTRIPE)],
                    tbl_s.at[pl.ds(sid * NSTRIPE, NSTRIPE)])
    pltpu.sync_copy(zd_hbm.at[pl.ds(sid * NSTRIPE, NSTRIPE)],
                    tbl_z.at[pl.ds(sid * NSTRIPE, NSTRIPE)])
    plsc.subcore_barrier()

    def run(table_sh, idx2d_hbm, out_hbm):
        pltpu.sync_copy(idx2d_hbm.at[pl.ds(row0, ROWS_PER_W)], idxall_v)

        def fire_gather(step, b):
            pltpu.async_copy(table_sh.at[idxall_v.at[step]],
                             rows_v.at[b], semg.at[b])

        def wait_gather(b):
            pltpu.make_async_copy(srcg_hbm.at[pl.ds(0, IDXB)],
                                  rows_v.at[b], semg.at[b]).wait()

        def fire_wb(step, b):
            pltpu.async_copy(rows_v.at[b],
                             out_hbm.at[pl.ds(e0 + step * IDXB, IDXB)],
                             semw.at[b])

        def wait_wb(b):
            pltpu.make_async_copy(srcg_hbm.at[pl.ds(0, IDXB)],
                                  rows_v.at[b], semw.at[b]).wait()

        for b in range(KBUF):
            fire_gather(b, b)

        def body(k, carry):
            for j in range(KBUF):
                s = k * KBUF + j
                wait_gather(j)
                fire_wb(s, j)
                jp = (j - 1) % KBUF
                sp = s - 1

                @pl.when((sp >= 0) & (sp + KBUF < ROWS_PER_W))
                def _():
                    wait_wb(jp)
                    fire_gather(sp + KBUF, jp)
            return carry
        lax.fori_loop(0, ROWS_PER_W // KBUF, body, 0)

        for b in range(KBUF):
            wait_wb(b)

    run(tbl_s, sidx_hbm, srcg_hbm)
    run(tbl_z, didx_hbm, dstg_hbm)


@functools.cache
def _sc_gather_kernel():
    return pl.kernel(
        _gather_kernel_body,
        out_type=[
            jax.ShapeDtypeStruct((E_PAD, D), jnp.bfloat16),
            jax.ShapeDtypeStruct((E_PAD, D), jnp.bfloat16),
        ],
        mesh=_sc_mesh(),
        scratch_types=[
            pltpu.VMEM((ROWS_PER_W, IDXB), jnp.int32),
            pltpu.VMEM((KBUF, IDXB, D), jnp.bfloat16),
            pltpu.VMEM_SHARED((N, D), jnp.bfloat16),
            pltpu.VMEM_SHARED((N, D), jnp.bfloat16),
            pltpu.SemaphoreType.DMA((KBUF,)),
            pltpu.SemaphoreType.DMA((KBUF,)),
        ],
        compiler_params=pltpu.CompilerParams(use_tc_tiling_on_sc=False),
    )


def _sc_gather(s, zd, sidx, didx):
    return _sc_gather_kernel()(s, zd, sidx, didx)


def _scatter_kernel_body(eout_hbm, didx_hbm, out0_hbm, out1_hbm, idx_v,
                         vals_v, zbuf_v, acc_shared):
    cid = lax.axis_index("c")
    sid = lax.axis_index("s")
    wid = sid * NC + cid
    row0 = wid * ROWS_PER_W
    e0 = wid * EDGES_PER_W

    def zrow(i, carry):
        for j in range(D // 16):
            zbuf_v[i, pl.ds(j * 16, 16)] = jnp.zeros((16,), jnp.float32)
        return carry
    lax.fori_loop(0, ZROWS, zrow, 0)
    for t in range(NPW // ZROWS):
        pltpu.sync_copy(zbuf_v,
                        acc_shared.at[pl.ds(sid * NPW + t * ZROWS, ZROWS)])
    plsc.subcore_barrier()

    def group(g, carry):
        r = row0 + g * SGROUP_ROWS
        pltpu.sync_copy(didx_hbm.at[pl.ds(r, SGROUP_ROWS)], idx_v)
        pltpu.sync_copy(eout_hbm.at[pl.ds(e0 + g * SGROUP_E, SGROUP_E)],
                        vals_v)
        for j in range(SGROUP_ROWS):
            pltpu.sync_copy(vals_v.at[pl.ds(j * IDXB, IDXB)],
                            acc_shared.at[idx_v.at[j]], add=True)
        return carry
    lax.fori_loop(0, SN_GROUPS, group, 0)
    plsc.subcore_barrier()

    @pl.when(cid == 0)
    def _():
        pltpu.sync_copy(acc_shared.at[pl.ds(sid * NPW, NPW)],
                        out0_hbm.at[pl.ds(sid * NPW, NPW)])

    @pl.when(cid == 1)
    def _():
        pltpu.sync_copy(acc_shared.at[pl.ds(sid * NPW, NPW)],
                        out1_hbm.at[pl.ds(sid * NPW, NPW)])


@functools.cache
def _sc_scatter_kernel():
    return pl.kernel(
        _scatter_kernel_body,
        out_type=[
            jax.ShapeDtypeStruct((N_PAD, D), jnp.float32),
            jax.ShapeDtypeStruct((N_PAD, D), jnp.float32),
        ],
        mesh=_sc_mesh(),
        scratch_types=[
            pltpu.VMEM((SGROUP_ROWS, IDXB), jnp.int32),
            pltpu.VMEM((SGROUP_E, D), jnp.float32),
            pltpu.VMEM((ZROWS, D), jnp.float32),
            pltpu.VMEM_SHARED((N_PAD, D), jnp.float32),
        ],
        compiler_params=pltpu.CompilerParams(use_tc_tiling_on_sc=False),
    )


def _sc_scatter(eout, didx):
    return _sc_scatter_kernel()(eout, didx)


def kernel(node_states, edge_index, rel_idx, rel_emb, msg_W1, msg_b1,
           msg_W2, msg_b2, gate_W1, gate_b1, gate_W2, gate_b2,
           ln_gamma, ln_beta):
    L = msg_W1.shape[0]

    pad = E_PAD - E
    sidx = jnp.pad(edge_index[0], (0, pad)).reshape(E_PAD // IDXB, IDXB)
    didx = jnp.pad(edge_index[1], (0, pad)).reshape(E_PAD // IDXB, IDXB)
    rel3d = jnp.pad(rel_idx, (0, pad)).reshape(E_PAD // TE, 1, TE)

    h = node_states
    for l in range(L):
        wa = msg_W1[l, :D]
        ga = gate_W1[l, :D]
        rel1 = rel_emb @ msg_W1[l, D:] + msg_b1[l]
        relg = rel_emb @ gate_W1[l, 2 * D:] + gate_b1[l]
        reltab = jnp.zeros((8, 2 * D), jnp.float32)
        reltab = reltab.at[:NREL, :D].set(rel1).at[:NREL, D:].set(relg)
        w2 = msg_W2[l]
        b2 = msg_b2[l][None, :]
        g1b = gate_W1[l, D:2 * D]
        g2row = gate_W2[l][:, 0][None, :]
        gb2 = gate_b2[l][None, :]

        s, zd = _node_transform(h, wa, ga)
        src_g, dst_g = _sc_gather(s, zd, sidx, didx)
        eout = _edge_mlp(src_g, dst_g, rel3d, reltab, w2, b2, g1b, g2row,
                         gb2)
        p0, p1 = _sc_scatter(eout, didx)
        h = _residual_ln(h, p0, p1, ln_gamma[l][None, :],
                         ln_beta[l][None, :])
    return h


# speedup vs baseline: 1.9543x; 1.4604x over previous
import functools

import jax
import jax.numpy as jnp
from jax import lax
from jax.experimental import pallas as pl
from jax.experimental.pallas import tpu as pltpu
from jax.experimental.pallas import tpu_sc as plsc

N = 10000
D = 128
E = 320000
NREL = 6

NC = 2
NS = 16
NW = NC * NS
IDXB = 128
ROWS_PER_W = 80
EDGES_PER_W = ROWS_PER_W * IDXB
E_PAD = NW * EDGES_PER_W
GROUP_ROWS = 4
GROUP_E = GROUP_ROWS * IDXB
N_GROUPS = ROWS_PER_W // GROUP_ROWS
SGROUP_ROWS = 2
SGROUP_E = SGROUP_ROWS * IDXB
SN_GROUPS = ROWS_PER_W // SGROUP_ROWS
ZROWS = 64

TE = 2048
TN = 2000
N_PAD = 10240
NPW = N_PAD // NS


def _pack_bf16(x):
    u = lax.bitcast_convert_type(x.astype(jnp.bfloat16),
                                 jnp.uint16).astype(jnp.uint32)
    w = u[:, :D // 2] | (u[:, D // 2:] << 16)
    return lax.bitcast_convert_type(w, jnp.float32)


def _unpack_bf16(x):
    w = lax.bitcast_convert_type(x, jnp.uint32)
    lo = lax.bitcast_convert_type((w & 0xFFFF).astype(jnp.uint16),
                                  jnp.bfloat16).astype(jnp.float32)
    hi = lax.bitcast_convert_type((w >> 16).astype(jnp.uint16),
                                  jnp.bfloat16).astype(jnp.float32)
    return jnp.concatenate([lo, hi], axis=1)


def _node_transform_body(h_ref, wa_ref, ga_ref, s_ref, zd_ref):
    x = h_ref[...]
    s_ref[...] = _pack_bf16(
        jnp.dot(x, wa_ref[...], preferred_element_type=jnp.float32))
    zd_ref[...] = _pack_bf16(
        jnp.dot(x, ga_ref[...], preferred_element_type=jnp.float32))


def _node_transform(h, wa, ga):
    return pl.pallas_call(
        _node_transform_body,
        grid=(N // TN,),
        in_specs=[
            pl.BlockSpec((TN, D), lambda i: (i, 0)),
            pl.BlockSpec((D, D), lambda i: (0, 0)),
            pl.BlockSpec((D, D), lambda i: (0, 0)),
        ],
        out_specs=[
            pl.BlockSpec((TN, D // 2), lambda i: (i, 0)),
            pl.BlockSpec((TN, D // 2), lambda i: (i, 0)),
        ],
        out_shape=[
            jax.ShapeDtypeStruct((N, D // 2), jnp.float32),
            jax.ShapeDtypeStruct((N, D // 2), jnp.float32),
        ],
    )(h, wa, ga)


def _gelu(x):
    return 0.5 * x * (1.0 + lax.erf(x * 0.7071067811865476))


def _edge_mlp_body(src_ref, dst_ref, rel_ref, reltab_ref, w2_ref, b2_ref,
                   g1b_ref, g2_ref, gb2_ref, out_ref):
    pid = pl.program_id(0)
    rel = rel_ref[0, 0, :]
    onehot = jnp.where(
        lax.broadcasted_iota(jnp.int32, (TE, 8), 1) == rel[:, None],
        1.0, 0.0).astype(jnp.float32)
    relb = jnp.dot(onehot, reltab_ref[...],
                   preferred_element_type=jnp.float32)
    hid = _gelu(_unpack_bf16(src_ref[...]) + relb[:, :D])
    msg = jnp.dot(hid, w2_ref[...],
                  preferred_element_type=jnp.float32) + b2_ref[...]
    g1 = _unpack_bf16(dst_ref[...]) \
        + jnp.dot(msg, g1b_ref[...], preferred_element_type=jnp.float32) \
        + relb[:, D:]
    ghid = _gelu(g1)
    glog = jnp.sum(ghid * g2_ref[...], axis=1, keepdims=True) + gb2_ref[0, 0]
    out = jax.nn.sigmoid(glog) * msg
    eid = pid * TE + lax.broadcasted_iota(jnp.int32, (TE, 1), 0)
    out_ref[...] = jnp.where(eid < E, out, 0.0)


def _edge_mlp(src_g, dst_g, rel3d, reltab, w2, b2, g1b, g2row, gb2):
    return pl.pallas_call(
        _edge_mlp_body,
        grid=(E_PAD // TE,),
        in_specs=[
            pl.BlockSpec((TE, D // 2), lambda i: (i, 0)),
            pl.BlockSpec((TE, D // 2), lambda i: (i, 0)),
            pl.BlockSpec((1, 1, TE), lambda i: (i, 0, 0)),
            pl.BlockSpec((8, 2 * D), lambda i: (0, 0)),
            pl.BlockSpec((D, D), lambda i: (0, 0)),
            pl.BlockSpec((1, D), lambda i: (0, 0)),
            pl.BlockSpec((D, D), lambda i: (0, 0)),
            pl.BlockSpec((1, D), lambda i: (0, 0)),
            pl.BlockSpec((1, 1), lambda i: (0, 0), memory_space=pltpu.SMEM),
        ],
        out_specs=pl.BlockSpec((TE, D), lambda i: (i, 0)),
        out_shape=jax.ShapeDtypeStruct((E_PAD, D), jnp.float32),
    )(src_g, dst_g, rel3d, reltab, w2, b2, g1b, g2row, gb2)


def _ln_body(h_ref, p0_ref, p1_ref, g_ref, b_ref, out_ref):
    x = h_ref[...] + p0_ref[...] + p1_ref[...]
    m = jnp.mean(x, axis=-1, keepdims=True)
    xc = x - m
    v = jnp.mean(xc * xc, axis=-1, keepdims=True)
    out_ref[...] = xc * lax.rsqrt(v + 1e-5) * g_ref[...] + b_ref[...]


def _residual_ln(h, p0, p1, gamma, beta):
    return pl.pallas_call(
        _ln_body,
        grid=(N // TN,),
        in_specs=[
            pl.BlockSpec((TN, D), lambda i: (i, 0)),
            pl.BlockSpec((TN, D), lambda i: (i, 0)),
            pl.BlockSpec((TN, D), lambda i: (i, 0)),
            pl.BlockSpec((1, D), lambda i: (0, 0)),
            pl.BlockSpec((1, D), lambda i: (0, 0)),
        ],
        out_specs=pl.BlockSpec((TN, D), lambda i: (i, 0)),
        out_shape=jax.ShapeDtypeStruct((N, D), jnp.float32),
    )(h, p0, p1, gamma, beta)


@functools.cache
def _sc_mesh():
    return plsc.VectorSubcoreMesh(core_axis_name="c", subcore_axis_name="s",
                                  num_cores=NC, num_subcores=NS)


KBUF = 4
NSTRIPE = N // NS


def _gather_kernel_body(s_hbm, zd_hbm, sidx_hbm, didx_hbm, srcg_hbm,
                        dstg_hbm, idxall_v, rows_v, tbl_s, tbl_z,
                        semg, semw):
    cid = lax.axis_index("c")
    sid = lax.axis_index("s")
    row0 = (cid * NS + sid) * ROWS_PER_W
    e0 = row0 * IDXB

    pltpu.sync_copy(s_hbm.at[pl.ds(sid * NSTRIPE, NSTRIPE)],
                    tbl_s.at[pl.ds(sid * NSTRIPE, NSTRIPE)])
    pltpu.sync_copy(zd_hbm.at[pl.ds(sid * NSTRIPE, NSTRIPE)],
                    tbl_z.at[pl.ds(sid * NSTRIPE, NSTRIPE)])
    plsc.subcore_barrier()

    def run(table_sh, idx2d_hbm, out_hbm):
        pltpu.sync_copy(idx2d_hbm.at[pl.ds(row0, ROWS_PER_W)], idxall_v)

        def fire_gather(step, b):
            pltpu.async_copy(table_sh.at[idxall_v.at[step]],
                             rows_v.at[b], semg.at[b])

        def wait_gather(b):
            pltpu.make_async_copy(srcg_hbm.at[pl.ds(0, IDXB)],
                                  rows_v.at[b], semg.at[b]).wait()

        def fire_wb(step, b):
            pltpu.async_copy(rows_v.at[b],
                             out_hbm.at[pl.ds(e0 + step * IDXB, IDXB)],
                             semw.at[b])

        def wait_wb(b):
            pltpu.make_async_copy(srcg_hbm.at[pl.ds(0, IDXB)],
                                  rows_v.at[b], semw.at[b]).wait()

        for b in range(KBUF):
            fire_gather(b, b)

        def body(k, carry):
            for j in range(KBUF):
                s = k * KBUF + j
                wait_gather(j)
                fire_wb(s, j)
                jp = (j - 1) % KBUF
                sp = s - 1

                @pl.when((sp >= 0) & (sp + KBUF < ROWS_PER_W))
                def _():
                    wait_wb(jp)
                    fire_gather(sp + KBUF, jp)
            return carry
        lax.fori_loop(0, ROWS_PER_W // KBUF, body, 0)

        for b in range(KBUF):
            wait_wb(b)

    run(tbl_s, sidx_hbm, srcg_hbm)
    run(tbl_z, didx_hbm, dstg_hbm)


@functools.cache
def _sc_gather_kernel():
    return pl.kernel(
        _gather_kernel_body,
        out_type=[
            jax.ShapeDtypeStruct((E_PAD, D // 2), jnp.float32),
            jax.ShapeDtypeStruct((E_PAD, D // 2), jnp.float32),
        ],
        mesh=_sc_mesh(),
        scratch_types=[
            pltpu.VMEM((ROWS_PER_W, IDXB), jnp.int32),
            pltpu.VMEM((KBUF, IDXB, D // 2), jnp.float32),
            pltpu.VMEM_SHARED((N, D // 2), jnp.float32),
            pltpu.VMEM_SHARED((N, D // 2), jnp.float32),
            pltpu.SemaphoreType.DMA((KBUF,)),
            pltpu.SemaphoreType.DMA((KBUF,)),
        ],
        compiler_params=pltpu.CompilerParams(use_tc_tiling_on_sc=False),
    )


def _sc_gather(s, zd, sidx, didx):
    return _sc_gather_kernel()(s, zd, sidx, didx)


def _scatter_kernel_body(eout_hbm, didx_hbm, out0_hbm, out1_hbm, idx_v,
                         vals_v, zbuf_v, acc_shared):
    cid = lax.axis_index("c")
    sid = lax.axis_index("s")
    wid = sid * NC + cid
    row0 = wid * ROWS_PER_W
    e0 = wid * EDGES_PER_W

    def zrow(i, carry):
        for j in range(D // 16):
            zbuf_v[i, pl.ds(j * 16, 16)] = jnp.zeros((16,), jnp.float32)
        return carry
    lax.fori_loop(0, ZROWS, zrow, 0)
    for t in range(NPW // ZROWS):
        pltpu.sync_copy(zbuf_v,
                        acc_shared.at[pl.ds(sid * NPW + t * ZROWS, ZROWS)])
    plsc.subcore_barrier()

    def group(g, carry):
        r = row0 + g * SGROUP_ROWS
        pltpu.sync_copy(didx_hbm.at[pl.ds(r, SGROUP_ROWS)], idx_v)
        pltpu.sync_copy(eout_hbm.at[pl.ds(e0 + g * SGROUP_E, SGROUP_E)],
                        vals_v)
        for j in range(SGROUP_ROWS):
            pltpu.sync_copy(vals_v.at[pl.ds(j * IDXB, IDXB)],
                            acc_shared.at[idx_v.at[j]], add=True)
        return carry
    lax.fori_loop(0, SN_GROUPS, group, 0)
    plsc.subcore_barrier()

    @pl.when(cid == 0)
    def _():
        pltpu.sync_copy(acc_shared.at[pl.ds(sid * NPW, NPW)],
                        out0_hbm.at[pl.ds(sid * NPW, NPW)])

    @pl.when(cid == 1)
    def _():
        pltpu.sync_copy(acc_shared.at[pl.ds(sid * NPW, NPW)],
                        out1_hbm.at[pl.ds(sid * NPW, NPW)])


@functools.cache
def _sc_scatter_kernel():
    return pl.kernel(
        _scatter_kernel_body,
        out_type=[
            jax.ShapeDtypeStruct((N_PAD, D), jnp.float32),
            jax.ShapeDtypeStruct((N_PAD, D), jnp.float32),
        ],
        mesh=_sc_mesh(),
        scratch_types=[
            pltpu.VMEM((SGROUP_ROWS, IDXB), jnp.int32),
            pltpu.VMEM((SGROUP_E, D), jnp.float32),
            pltpu.VMEM((ZROWS, D), jnp.float32),
            pltpu.VMEM_SHARED((N_PAD, D), jnp.float32),
        ],
        compiler_params=pltpu.CompilerParams(use_tc_tiling_on_sc=False),
    )


def _sc_scatter(eout, didx):
    return _sc_scatter_kernel()(eout, didx)


def kernel(node_states, edge_index, rel_idx, rel_emb, msg_W1, msg_b1,
           msg_W2, msg_b2, gate_W1, gate_b1, gate_W2, gate_b2,
           ln_gamma, ln_beta):
    L = msg_W1.shape[0]

    pad = E_PAD - E
    sidx = jnp.pad(edge_index[0], (0, pad)).reshape(E_PAD // IDXB, IDXB)
    didx = jnp.pad(edge_index[1], (0, pad)).reshape(E_PAD // IDXB, IDXB)
    rel3d = jnp.pad(rel_idx, (0, pad)).reshape(E_PAD // TE, 1, TE)

    h = node_states
    for l in range(L):
        wa = msg_W1[l, :D]
        ga = gate_W1[l, :D]
        rel1 = rel_emb @ msg_W1[l, D:] + msg_b1[l]
        relg = rel_emb @ gate_W1[l, 2 * D:] + gate_b1[l]
        reltab = jnp.zeros((8, 2 * D), jnp.float32)
        reltab = reltab.at[:NREL, :D].set(rel1).at[:NREL, D:].set(relg)
        w2 = msg_W2[l]
        b2 = msg_b2[l][None, :]
        g1b = gate_W1[l, D:2 * D]
        g2row = gate_W2[l][:, 0][None, :]
        gb2 = gate_b2[l][None, :]

        s, zd = _node_transform(h, wa, ga)
        src_g, dst_g = _sc_gather(s, zd, sidx, didx)
        eout = _edge_mlp(src_g, dst_g, rel3d, reltab, w2, b2, g1b, g2row,
                         gb2)
        p0, p1 = _sc_scatter(eout, didx)
        h = _residual_ln(h, p0, p1, ln_gamma[l][None, :],
                         ln_beta[l][None, :])
    return h


# speedup vs baseline: 1.9664x; 1.0062x over previous
import functools

import jax
import jax.numpy as jnp
from jax import lax
from jax.experimental import pallas as pl
from jax.experimental.pallas import tpu as pltpu
from jax.experimental.pallas import tpu_sc as plsc

N = 10000
D = 128
E = 320000
NREL = 6

NC = 2
NS = 16
NW = NC * NS
IDXB = 128
ROWS_PER_W = 80
EDGES_PER_W = ROWS_PER_W * IDXB
E_PAD = NW * EDGES_PER_W
GROUP_ROWS = 4
GROUP_E = GROUP_ROWS * IDXB
N_GROUPS = ROWS_PER_W // GROUP_ROWS
SGROUP_ROWS = 2
SGROUP_E = SGROUP_ROWS * IDXB
SN_GROUPS = ROWS_PER_W // SGROUP_ROWS
ZROWS = 64

TE = 2048
TN = 2000
N_PAD = 10240
NPW = N_PAD // NS


def _pack_bf16(x):
    u = lax.bitcast_convert_type(x.astype(jnp.bfloat16),
                                 jnp.uint16).astype(jnp.uint32)
    w = u[:, :D // 2] | (u[:, D // 2:] << 16)
    return lax.bitcast_convert_type(w, jnp.float32)


def _unpack_bf16(x):
    w = lax.bitcast_convert_type(x, jnp.uint32)
    lo = lax.bitcast_convert_type((w & 0xFFFF).astype(jnp.uint16),
                                  jnp.bfloat16).astype(jnp.float32)
    hi = lax.bitcast_convert_type((w >> 16).astype(jnp.uint16),
                                  jnp.bfloat16).astype(jnp.float32)
    return jnp.concatenate([lo, hi], axis=1)


def _node_transform_body(h_ref, wa_ref, ga_ref, s_ref, zd_ref):
    x = h_ref[...]
    s_ref[...] = _pack_bf16(
        jnp.dot(x, wa_ref[...], preferred_element_type=jnp.float32))
    zd_ref[...] = _pack_bf16(
        jnp.dot(x, ga_ref[...], preferred_element_type=jnp.float32))


def _node_transform(h, wa, ga):
    return pl.pallas_call(
        _node_transform_body,
        grid=(N // TN,),
        in_specs=[
            pl.BlockSpec((TN, D), lambda i: (i, 0)),
            pl.BlockSpec((D, D), lambda i: (0, 0)),
            pl.BlockSpec((D, D), lambda i: (0, 0)),
        ],
        out_specs=[
            pl.BlockSpec((TN, D // 2), lambda i: (i, 0)),
            pl.BlockSpec((TN, D // 2), lambda i: (i, 0)),
        ],
        out_shape=[
            jax.ShapeDtypeStruct((N, D // 2), jnp.float32),
            jax.ShapeDtypeStruct((N, D // 2), jnp.float32),
        ],
    )(h, wa, ga)


def _gelu(x):
    return 0.5 * x * (1.0 + lax.erf(x * 0.7071067811865476))


def _edge_mlp_body(src_hbm, dst_hbm, rel_ref, reltab_ref, w2_ref, b2_ref,
                   g1b_ref, g2_ref, gb2_ref, out_ref,
                   src_v, dst_v, sems, semd):
    pid = pl.program_id(0)
    nb = pl.num_programs(0)

    def fetch(i, slot):
        pltpu.make_async_copy(src_hbm.at[pl.ds(i * TE, TE)], src_v.at[slot],
                              sems.at[slot]).start()
        pltpu.make_async_copy(dst_hbm.at[pl.ds(i * TE, TE)], dst_v.at[slot],
                              semd.at[slot]).start()

    @pl.when(pid == 0)
    def _():
        fetch(0, 0)

    @pl.when(pid + 1 < nb)
    def _():
        fetch(pid + 1, (pid + 1) % 2)

    slot = pid % 2
    pltpu.make_async_copy(src_hbm.at[pl.ds(0, TE)], src_v.at[slot],
                          sems.at[slot]).wait()
    pltpu.make_async_copy(dst_hbm.at[pl.ds(0, TE)], dst_v.at[slot],
                          semd.at[slot]).wait()

    src = _unpack_bf16(src_v[slot])
    dst = _unpack_bf16(dst_v[slot])
    rel = rel_ref[0, 0, :]
    onehot = jnp.where(
        lax.broadcasted_iota(jnp.int32, (TE, 8), 1) == rel[:, None],
        1.0, 0.0).astype(jnp.float32)
    relb = jnp.dot(onehot, reltab_ref[...],
                   preferred_element_type=jnp.float32)
    hid = _gelu(src + relb[:, :D])
    msg = jnp.dot(hid, w2_ref[...],
                  preferred_element_type=jnp.float32) + b2_ref[...]
    g1 = dst \
        + jnp.dot(msg, g1b_ref[...], preferred_element_type=jnp.float32) \
        + relb[:, D:]
    ghid = _gelu(g1)
    glog = jnp.sum(ghid * g2_ref[...], axis=1, keepdims=True) + gb2_ref[0, 0]
    out = jax.nn.sigmoid(glog) * msg
    eid = pid * TE + lax.broadcasted_iota(jnp.int32, (TE, 1), 0)
    out_ref[...] = jnp.where(eid < E, out, 0.0)


def _edge_mlp(src_g, dst_g, rel3d, reltab, w2, b2, g1b, g2row, gb2):
    return pl.pallas_call(
        _edge_mlp_body,
        grid=(E_PAD // TE,),
        in_specs=[
            pl.BlockSpec(memory_space=pl.ANY),
            pl.BlockSpec(memory_space=pl.ANY),
            pl.BlockSpec((1, 1, TE), lambda i: (i, 0, 0)),
            pl.BlockSpec((8, 2 * D), lambda i: (0, 0)),
            pl.BlockSpec((D, D), lambda i: (0, 0)),
            pl.BlockSpec((1, D), lambda i: (0, 0)),
            pl.BlockSpec((D, D), lambda i: (0, 0)),
            pl.BlockSpec((1, D), lambda i: (0, 0)),
            pl.BlockSpec((1, 1), lambda i: (0, 0), memory_space=pltpu.SMEM),
        ],
        out_specs=pl.BlockSpec((TE, D), lambda i: (i, 0)),
        out_shape=jax.ShapeDtypeStruct((E_PAD, D), jnp.float32),
        scratch_shapes=[
            pltpu.VMEM((2, TE, D // 2), jnp.float32),
            pltpu.VMEM((2, TE, D // 2), jnp.float32),
            pltpu.SemaphoreType.DMA((2,)),
            pltpu.SemaphoreType.DMA((2,)),
        ],
    )(src_g, dst_g, rel3d, reltab, w2, b2, g1b, g2row, gb2)


def _ln_body(h_ref, p0_ref, p1_ref, g_ref, b_ref, out_ref):
    x = h_ref[...] + p0_ref[...] + p1_ref[...]
    m = jnp.mean(x, axis=-1, keepdims=True)
    xc = x - m
    v = jnp.mean(xc * xc, axis=-1, keepdims=True)
    out_ref[...] = xc * lax.rsqrt(v + 1e-5) * g_ref[...] + b_ref[...]


def _residual_ln(h, p0, p1, gamma, beta):
    return pl.pallas_call(
        _ln_body,
        grid=(N // TN,),
        in_specs=[
            pl.BlockSpec((TN, D), lambda i: (i, 0)),
            pl.BlockSpec((TN, D), lambda i: (i, 0)),
            pl.BlockSpec((TN, D), lambda i: (i, 0)),
            pl.BlockSpec((1, D), lambda i: (0, 0)),
            pl.BlockSpec((1, D), lambda i: (0, 0)),
        ],
        out_specs=pl.BlockSpec((TN, D), lambda i: (i, 0)),
        out_shape=jax.ShapeDtypeStruct((N, D), jnp.float32),
    )(h, p0, p1, gamma, beta)


@functools.cache
def _sc_mesh():
    return plsc.VectorSubcoreMesh(core_axis_name="c", subcore_axis_name="s",
                                  num_cores=NC, num_subcores=NS)


KBUF = 4
NSTRIPE = N // NS


def _gather_kernel_body(s_hbm, zd_hbm, sidx_hbm, didx_hbm, srcg_hbm,
                        dstg_hbm, idxall_v, rows_v, tbl_s, tbl_z,
                        semg, semw):
    cid = lax.axis_index("c")
    sid = lax.axis_index("s")
    row0 = (cid * NS + sid) * ROWS_PER_W
    e0 = row0 * IDXB

    pltpu.sync_copy(s_hbm.at[pl.ds(sid * NSTRIPE, NSTRIPE)],
                    tbl_s.at[pl.ds(sid * NSTRIPE, NSTRIPE)])
    pltpu.sync_copy(zd_hbm.at[pl.ds(sid * NSTRIPE, NSTRIPE)],
                    tbl_z.at[pl.ds(sid * NSTRIPE, NSTRIPE)])
    plsc.subcore_barrier()

    def run(table_sh, idx2d_hbm, out_hbm):
        pltpu.sync_copy(idx2d_hbm.at[pl.ds(row0, ROWS_PER_W)], idxall_v)

        def fire_gather(step, b):
            pltpu.async_copy(table_sh.at[idxall_v.at[step]],
                             rows_v.at[b], semg.at[b])

        def wait_gather(b):
            pltpu.make_async_copy(srcg_hbm.at[pl.ds(0, IDXB)],
                                  rows_v.at[b], semg.at[b]).wait()

        def fire_wb(step, b):
            pltpu.async_copy(rows_v.at[b],
                             out_hbm.at[pl.ds(e0 + step * IDXB, IDXB)],
                             semw.at[b])

        def wait_wb(b):
            pltpu.make_async_copy(srcg_hbm.at[pl.ds(0, IDXB)],
                                  rows_v.at[b], semw.at[b]).wait()

        for b in range(KBUF):
            fire_gather(b, b)

        def body(k, carry):
            for j in range(KBUF):
                s = k * KBUF + j
                wait_gather(j)
                fire_wb(s, j)
                jp = (j - 1) % KBUF
                sp = s - 1

                @pl.when((sp >= 0) & (sp + KBUF < ROWS_PER_W))
                def _():
                    wait_wb(jp)
                    fire_gather(sp + KBUF, jp)
            return carry
        lax.fori_loop(0, ROWS_PER_W // KBUF, body, 0)

        for b in range(KBUF):
            wait_wb(b)

    run(tbl_s, sidx_hbm, srcg_hbm)
    run(tbl_z, didx_hbm, dstg_hbm)


@functools.cache
def _sc_gather_kernel():
    return pl.kernel(
        _gather_kernel_body,
        out_type=[
            jax.ShapeDtypeStruct((E_PAD, D // 2), jnp.float32),
            jax.ShapeDtypeStruct((E_PAD, D // 2), jnp.float32),
        ],
        mesh=_sc_mesh(),
        scratch_types=[
            pltpu.VMEM((ROWS_PER_W, IDXB), jnp.int32),
            pltpu.VMEM((KBUF, IDXB, D // 2), jnp.float32),
            pltpu.VMEM_SHARED((N, D // 2), jnp.float32),
            pltpu.VMEM_SHARED((N, D // 2), jnp.float32),
            pltpu.SemaphoreType.DMA((KBUF,)),
            pltpu.SemaphoreType.DMA((KBUF,)),
        ],
        compiler_params=pltpu.CompilerParams(use_tc_tiling_on_sc=False),
    )


def _sc_gather(s, zd, sidx, didx):
    return _sc_gather_kernel()(s, zd, sidx, didx)


def _scatter_kernel_body(eout_hbm, didx_hbm, out0_hbm, out1_hbm, idx_v,
                         vals_v, zbuf_v, acc_shared):
    cid = lax.axis_index("c")
    sid = lax.axis_index("s")
    wid = sid * NC + cid
    row0 = wid * ROWS_PER_W
    e0 = wid * EDGES_PER_W

    def zrow(i, carry):
        for j in range(D // 16):
            zbuf_v[i, pl.ds(j * 16, 16)] = jnp.zeros((16,), jnp.float32)
        return carry
    lax.fori_loop(0, ZROWS, zrow, 0)
    for t in range(NPW // ZROWS):
        pltpu.sync_copy(zbuf_v,
                        acc_shared.at[pl.ds(sid * NPW + t * ZROWS, ZROWS)])
    plsc.subcore_barrier()

    def group(g, carry):
        r = row0 + g * SGROUP_ROWS
        pltpu.sync_copy(didx_hbm.at[pl.ds(r, SGROUP_ROWS)], idx_v)
        pltpu.sync_copy(eout_hbm.at[pl.ds(e0 + g * SGROUP_E, SGROUP_E)],
                        vals_v)
        for j in range(SGROUP_ROWS):
            pltpu.sync_copy(vals_v.at[pl.ds(j * IDXB, IDXB)],
                            acc_shared.at[idx_v.at[j]], add=True)
        return carry
    lax.fori_loop(0, SN_GROUPS, group, 0)
    plsc.subcore_barrier()

    @pl.when(cid == 0)
    def _():
        pltpu.sync_copy(acc_shared.at[pl.ds(sid * NPW, NPW)],
                        out0_hbm.at[pl.ds(sid * NPW, NPW)])

    @pl.when(cid == 1)
    def _():
        pltpu.sync_copy(acc_shared.at[pl.ds(sid * NPW, NPW)],
                        out1_hbm.at[pl.ds(sid * NPW, NPW)])


@functools.cache
def _sc_scatter_kernel():
    return pl.kernel(
        _scatter_kernel_body,
        out_type=[
            jax.ShapeDtypeStruct((N_PAD, D), jnp.float32),
            jax.ShapeDtypeStruct((N_PAD, D), jnp.float32),
        ],
        mesh=_sc_mesh(),
        scratch_types=[
            pltpu.VMEM((SGROUP_ROWS, IDXB), jnp.int32),
            pltpu.VMEM((SGROUP_E, D), jnp.float32),
            pltpu.VMEM((ZROWS, D), jnp.float32),
            pltpu.VMEM_SHARED((N_PAD, D), jnp.float32),
        ],
        compiler_params=pltpu.CompilerParams(use_tc_tiling_on_sc=False),
    )


def _sc_scatter(eout, didx):
    return _sc_scatter_kernel()(eout, didx)


def kernel(node_states, edge_index, rel_idx, rel_emb, msg_W1, msg_b1,
           msg_W2, msg_b2, gate_W1, gate_b1, gate_W2, gate_b2,
           ln_gamma, ln_beta):
    L = msg_W1.shape[0]

    pad = E_PAD - E
    sidx = jnp.pad(edge_index[0], (0, pad)).reshape(E_PAD // IDXB, IDXB)
    didx = jnp.pad(edge_index[1], (0, pad)).reshape(E_PAD // IDXB, IDXB)
    rel3d = jnp.pad(rel_idx, (0, pad)).reshape(E_PAD // TE, 1, TE)

    h = node_states
    for l in range(L):
        wa = msg_W1[l, :D]
        ga = gate_W1[l, :D]
        rel1 = rel_emb @ msg_W1[l, D:] + msg_b1[l]
        relg = rel_emb @ gate_W1[l, 2 * D:] + gate_b1[l]
        reltab = jnp.zeros((8, 2 * D), jnp.float32)
        reltab = reltab.at[:NREL, :D].set(rel1).at[:NREL, D:].set(relg)
        w2 = msg_W2[l]
        b2 = msg_b2[l][None, :]
        g1b = gate_W1[l, D:2 * D]
        g2row = gate_W2[l][:, 0][None, :]
        gb2 = gate_b2[l][None, :]

        s, zd = _node_transform(h, wa, ga)
        src_g, dst_g = _sc_gather(s, zd, sidx, didx)
        eout = _edge_mlp(src_g, dst_g, rel3d, reltab, w2, b2, g1b, g2row,
                         gb2)
        p0, p1 = _sc_scatter(eout, didx)
        h = _residual_ln(h, p0, p1, ln_gamma[l][None, :],
                         ln_beta[l][None, :])
    return h


# speedup vs baseline: 2.1215x; 1.0789x over previous
import functools

import jax
import jax.numpy as jnp
from jax import lax
from jax.experimental import pallas as pl
from jax.experimental.pallas import tpu as pltpu
from jax.experimental.pallas import tpu_sc as plsc

N = 10000
D = 128
E = 320000
NREL = 6

NC = 2
NS = 16
NW = NC * NS
IDXB = 128
ROWS_PER_W = 80
EDGES_PER_W = ROWS_PER_W * IDXB
E_PAD = NW * EDGES_PER_W
GROUP_ROWS = 4
GROUP_E = GROUP_ROWS * IDXB
N_GROUPS = ROWS_PER_W // GROUP_ROWS
SGROUP_ROWS = 2
SGROUP_E = SGROUP_ROWS * IDXB
SN_GROUPS = ROWS_PER_W // SGROUP_ROWS
ZROWS = 32

TE = 2048
TN = 2000
N_PAD = 10240
NPW = N_PAD // NS


def _pack_bf16(x):
    u = lax.bitcast_convert_type(x.astype(jnp.bfloat16),
                                 jnp.uint16).astype(jnp.uint32)
    w = u[:, :D // 2] | (u[:, D // 2:] << 16)
    return lax.bitcast_convert_type(w, jnp.float32)


def _unpack_bf16(x):
    w = lax.bitcast_convert_type(x, jnp.uint32)
    lo = lax.bitcast_convert_type((w & 0xFFFF).astype(jnp.uint16),
                                  jnp.bfloat16).astype(jnp.float32)
    hi = lax.bitcast_convert_type((w >> 16).astype(jnp.uint16),
                                  jnp.bfloat16).astype(jnp.float32)
    return jnp.concatenate([lo, hi], axis=1)


def _node_transform_body(h_ref, wa_ref, ga_ref, s_ref, zd_ref):
    x = h_ref[...]
    s_ref[...] = _pack_bf16(
        jnp.dot(x, wa_ref[...], preferred_element_type=jnp.float32))
    zd_ref[...] = _pack_bf16(
        jnp.dot(x, ga_ref[...], preferred_element_type=jnp.float32))


def _node_transform(h, wa, ga):
    return pl.pallas_call(
        _node_transform_body,
        grid=(N // TN,),
        in_specs=[
            pl.BlockSpec((TN, D), lambda i: (i, 0)),
            pl.BlockSpec((D, D), lambda i: (0, 0)),
            pl.BlockSpec((D, D), lambda i: (0, 0)),
        ],
        out_specs=[
            pl.BlockSpec((TN, D // 2), lambda i: (i, 0)),
            pl.BlockSpec((TN, D // 2), lambda i: (i, 0)),
        ],
        out_shape=[
            jax.ShapeDtypeStruct((N, D // 2), jnp.float32),
            jax.ShapeDtypeStruct((N, D // 2), jnp.float32),
        ],
    )(h, wa, ga)


def _gelu(x):
    return 0.5 * x * (1.0 + lax.erf(x * 0.7071067811865476))


def _edge_mlp_body(src_hbm, dst_hbm, rel_ref, reltab_ref, w2_ref, b2_ref,
                   g1b_ref, g2_ref, gb2_ref, out_ref,
                   src_v, dst_v, sems, semd):
    pid = pl.program_id(0)
    nb = pl.num_programs(0)

    def fetch(i, slot):
        pltpu.make_async_copy(src_hbm.at[pl.ds(i * TE, TE)], src_v.at[slot],
                              sems.at[slot]).start()
        pltpu.make_async_copy(dst_hbm.at[pl.ds(i * TE, TE)], dst_v.at[slot],
                              semd.at[slot]).start()

    @pl.when(pid == 0)
    def _():
        fetch(0, 0)

    @pl.when(pid + 1 < nb)
    def _():
        fetch(pid + 1, (pid + 1) % 2)

    slot = pid % 2
    pltpu.make_async_copy(src_hbm.at[pl.ds(0, TE)], src_v.at[slot],
                          sems.at[slot]).wait()
    pltpu.make_async_copy(dst_hbm.at[pl.ds(0, TE)], dst_v.at[slot],
                          semd.at[slot]).wait()

    src = _unpack_bf16(src_v[slot])
    dst = _unpack_bf16(dst_v[slot])
    rel = rel_ref[0, 0, :]
    onehot = jnp.where(
        lax.broadcasted_iota(jnp.int32, (TE, 8), 1) == rel[:, None],
        1.0, 0.0).astype(jnp.float32)
    relb = jnp.dot(onehot, reltab_ref[...],
                   preferred_element_type=jnp.float32)
    hid = _gelu(src + relb[:, :D])
    msg = jnp.dot(hid, w2_ref[...],
                  preferred_element_type=jnp.float32) + b2_ref[...]
    g1 = dst \
        + jnp.dot(msg, g1b_ref[...], preferred_element_type=jnp.float32) \
        + relb[:, D:]
    ghid = _gelu(g1)
    glog = jnp.sum(ghid * g2_ref[...], axis=1, keepdims=True) + gb2_ref[0, 0]
    out = jax.nn.sigmoid(glog) * msg
    eid = pid * TE + lax.broadcasted_iota(jnp.int32, (TE, 1), 0)
    out_ref[...] = jnp.where(eid < E, out, 0.0)


def _edge_mlp(src_g, dst_g, rel3d, reltab, w2, b2, g1b, g2row, gb2):
    return pl.pallas_call(
        _edge_mlp_body,
        grid=(E_PAD // TE,),
        in_specs=[
            pl.BlockSpec(memory_space=pl.ANY),
            pl.BlockSpec(memory_space=pl.ANY),
            pl.BlockSpec((1, 1, TE), lambda i: (i, 0, 0)),
            pl.BlockSpec((8, 2 * D), lambda i: (0, 0)),
            pl.BlockSpec((D, D), lambda i: (0, 0)),
            pl.BlockSpec((1, D), lambda i: (0, 0)),
            pl.BlockSpec((D, D), lambda i: (0, 0)),
            pl.BlockSpec((1, D), lambda i: (0, 0)),
            pl.BlockSpec((1, 1), lambda i: (0, 0), memory_space=pltpu.SMEM),
        ],
        out_specs=pl.BlockSpec((TE, D), lambda i: (i, 0)),
        out_shape=jax.ShapeDtypeStruct((E_PAD, D), jnp.float32),
        scratch_shapes=[
            pltpu.VMEM((2, TE, D // 2), jnp.float32),
            pltpu.VMEM((2, TE, D // 2), jnp.float32),
            pltpu.SemaphoreType.DMA((2,)),
            pltpu.SemaphoreType.DMA((2,)),
        ],
    )(src_g, dst_g, rel3d, reltab, w2, b2, g1b, g2row, gb2)


def _ln_body(h_ref, p0_ref, p1_ref, g_ref, b_ref, out_ref):
    x = h_ref[...] + p0_ref[...] + p1_ref[...]
    m = jnp.mean(x, axis=-1, keepdims=True)
    xc = x - m
    v = jnp.mean(xc * xc, axis=-1, keepdims=True)
    out_ref[...] = xc * lax.rsqrt(v + 1e-5) * g_ref[...] + b_ref[...]


def _residual_ln(h, p0, p1, gamma, beta):
    return pl.pallas_call(
        _ln_body,
        grid=(N // TN,),
        in_specs=[
            pl.BlockSpec((TN, D), lambda i: (i, 0)),
            pl.BlockSpec((TN, D), lambda i: (i, 0)),
            pl.BlockSpec((TN, D), lambda i: (i, 0)),
            pl.BlockSpec((1, D), lambda i: (0, 0)),
            pl.BlockSpec((1, D), lambda i: (0, 0)),
        ],
        out_specs=pl.BlockSpec((TN, D), lambda i: (i, 0)),
        out_shape=jax.ShapeDtypeStruct((N, D), jnp.float32),
    )(h, p0, p1, gamma, beta)


@functools.cache
def _sc_mesh():
    return plsc.VectorSubcoreMesh(core_axis_name="c", subcore_axis_name="s",
                                  num_cores=NC, num_subcores=NS)


KBUF = 4
NSTRIPE = N // NS


def _gather_kernel_body(s_hbm, zd_hbm, sidx_hbm, didx_hbm, srcg_hbm,
                        dstg_hbm, idxall_v, rows_v, tbl_s, tbl_z,
                        semg, semw):
    cid = lax.axis_index("c")
    sid = lax.axis_index("s")
    row0 = (cid * NS + sid) * ROWS_PER_W
    e0 = row0 * IDXB

    pltpu.sync_copy(s_hbm.at[pl.ds(sid * NSTRIPE, NSTRIPE)],
                    tbl_s.at[pl.ds(sid * NSTRIPE, NSTRIPE)])
    pltpu.sync_copy(zd_hbm.at[pl.ds(sid * NSTRIPE, NSTRIPE)],
                    tbl_z.at[pl.ds(sid * NSTRIPE, NSTRIPE)])
    plsc.subcore_barrier()

    def run(table_sh, idx2d_hbm, out_hbm):
        pltpu.sync_copy(idx2d_hbm.at[pl.ds(row0, ROWS_PER_W)], idxall_v)

        def fire_gather(step, b):
            pltpu.async_copy(table_sh.at[idxall_v.at[step]],
                             rows_v.at[b], semg.at[b])

        def wait_gather(b):
            pltpu.make_async_copy(srcg_hbm.at[pl.ds(0, IDXB)],
                                  rows_v.at[b], semg.at[b]).wait()

        def fire_wb(step, b):
            pltpu.async_copy(rows_v.at[b],
                             out_hbm.at[pl.ds(e0 + step * IDXB, IDXB)],
                             semw.at[b])

        def wait_wb(b):
            pltpu.make_async_copy(srcg_hbm.at[pl.ds(0, IDXB)],
                                  rows_v.at[b], semw.at[b]).wait()

        for b in range(KBUF):
            fire_gather(b, b)

        def body(k, carry):
            for j in range(KBUF):
                s = k * KBUF + j
                wait_gather(j)
                fire_wb(s, j)
                jp = (j - 1) % KBUF
                sp = s - 1

                @pl.when((sp >= 0) & (sp + KBUF < ROWS_PER_W))
                def _():
                    wait_wb(jp)
                    fire_gather(sp + KBUF, jp)
            return carry
        lax.fori_loop(0, ROWS_PER_W // KBUF, body, 0)

        for b in range(KBUF):
            wait_wb(b)

    run(tbl_s, sidx_hbm, srcg_hbm)
    run(tbl_z, didx_hbm, dstg_hbm)


@functools.cache
def _sc_gather_kernel():
    return pl.kernel(
        _gather_kernel_body,
        out_type=[
            jax.ShapeDtypeStruct((E_PAD, D // 2), jnp.float32),
            jax.ShapeDtypeStruct((E_PAD, D // 2), jnp.float32),
        ],
        mesh=_sc_mesh(),
        scratch_types=[
            pltpu.VMEM((ROWS_PER_W, IDXB), jnp.int32),
            pltpu.VMEM((KBUF, IDXB, D // 2), jnp.float32),
            pltpu.VMEM_SHARED((N, D // 2), jnp.float32),
            pltpu.VMEM_SHARED((N, D // 2), jnp.float32),
            pltpu.SemaphoreType.DMA((KBUF,)),
            pltpu.SemaphoreType.DMA((KBUF,)),
        ],
        compiler_params=pltpu.CompilerParams(use_tc_tiling_on_sc=False),
    )


def _sc_gather(s, zd, sidx, didx):
    return _sc_gather_kernel()(s, zd, sidx, didx)


def _scatter_kernel_body(eout_hbm, didx_hbm, out0_hbm, out1_hbm, idxall_v,
                         vals_v, zbuf_v, acc_shared, semv):
    cid = lax.axis_index("c")
    sid = lax.axis_index("s")
    wid = sid * NC + cid
    row0 = wid * ROWS_PER_W
    e0 = wid * EDGES_PER_W

    pltpu.async_copy(didx_hbm.at[pl.ds(row0, ROWS_PER_W)], idxall_v,
                     semv.at[0])

    def zrow(i, carry):
        for j in range(D // 16):
            zbuf_v[i, pl.ds(j * 16, 16)] = jnp.zeros((16,), jnp.float32)
        return carry
    lax.fori_loop(0, ZROWS, zrow, 0)
    for t in range(NPW // ZROWS):
        pltpu.sync_copy(zbuf_v,
                        acc_shared.at[pl.ds(sid * NPW + t * ZROWS, ZROWS)])
    pltpu.make_async_copy(didx_hbm.at[pl.ds(0, ROWS_PER_W)], idxall_v,
                          semv.at[0]).wait()
    plsc.subcore_barrier()

    def fire_load(step, b):
        pltpu.async_copy(eout_hbm.at[pl.ds(e0 + step * IDXB, IDXB)],
                         vals_v.at[b], semv.at[b])

    def wait_load(b):
        pltpu.make_async_copy(eout_hbm.at[pl.ds(0, IDXB)], vals_v.at[b],
                              semv.at[b]).wait()

    for b in range(2):
        fire_load(b, b)

    def group(k, carry):
        for j in range(2):
            s = 2 * k + j
            wait_load(j)
            pltpu.sync_copy(vals_v.at[j], acc_shared.at[idxall_v.at[s]],
                            add=True)

            @pl.when(s + 2 < ROWS_PER_W)
            def _():
                fire_load(s + 2, j)
        return carry
    lax.fori_loop(0, ROWS_PER_W // 2, group, 0)
    plsc.subcore_barrier()

    @pl.when(cid == 0)
    def _():
        pltpu.sync_copy(acc_shared.at[pl.ds(sid * NPW, NPW)],
                        out0_hbm.at[pl.ds(sid * NPW, NPW)])

    @pl.when(cid == 1)
    def _():
        pltpu.sync_copy(acc_shared.at[pl.ds(sid * NPW, NPW)],
                        out1_hbm.at[pl.ds(sid * NPW, NPW)])


@functools.cache
def _sc_scatter_kernel():
    return pl.kernel(
        _scatter_kernel_body,
        out_type=[
            jax.ShapeDtypeStruct((N_PAD, D), jnp.float32),
            jax.ShapeDtypeStruct((N_PAD, D), jnp.float32),
        ],
        mesh=_sc_mesh(),
        scratch_types=[
            pltpu.VMEM((ROWS_PER_W, IDXB), jnp.int32),
            pltpu.VMEM((2, IDXB, D), jnp.float32),
            pltpu.VMEM((ZROWS, D), jnp.float32),
            pltpu.VMEM_SHARED((N_PAD, D), jnp.float32),
            pltpu.SemaphoreType.DMA((2,)),
        ],
        compiler_params=pltpu.CompilerParams(use_tc_tiling_on_sc=False),
    )


def _sc_scatter(eout, didx):
    return _sc_scatter_kernel()(eout, didx)


def kernel(node_states, edge_index, rel_idx, rel_emb, msg_W1, msg_b1,
           msg_W2, msg_b2, gate_W1, gate_b1, gate_W2, gate_b2,
           ln_gamma, ln_beta):
    L = msg_W1.shape[0]

    pad = E_PAD - E
    sidx = jnp.pad(edge_index[0], (0, pad)).reshape(E_PAD // IDXB, IDXB)
    didx = jnp.pad(edge_index[1], (0, pad)).reshape(E_PAD // IDXB, IDXB)
    rel3d = jnp.pad(rel_idx, (0, pad)).reshape(E_PAD // TE, 1, TE)

    h = node_states
    for l in range(L):
        wa = msg_W1[l, :D]
        ga = gate_W1[l, :D]
        rel1 = rel_emb @ msg_W1[l, D:] + msg_b1[l]
        relg = rel_emb @ gate_W1[l, 2 * D:] + gate_b1[l]
        reltab = jnp.zeros((8, 2 * D), jnp.float32)
        reltab = reltab.at[:NREL, :D].set(rel1).at[:NREL, D:].set(relg)
        w2 = msg_W2[l]
        b2 = msg_b2[l][None, :]
        g1b = gate_W1[l, D:2 * D]
        g2row = gate_W2[l][:, 0][None, :]
        gb2 = gate_b2[l][None, :]

        s, zd = _node_transform(h, wa, ga)
        src_g, dst_g = _sc_gather(s, zd, sidx, didx)
        eout = _edge_mlp(src_g, dst_g, rel3d, reltab, w2, b2, g1b, g2row,
                         gb2)
        p0, p1 = _sc_scatter(eout, didx)
        h = _residual_ln(h, p0, p1, ln_gamma[l][None, :],
                         ln_beta[l][None, :])
    return h
